# Initial kernel scaffold; baseline (speedup 1.0000x reference)
import jax
import jax.numpy as jnp
from jax.experimental import pallas as pl


def kernel(x, edge_index, edge_weight, W1_1, W2_1, b1, W1_2, W2_2, b2, Wd, bd):
    raise NotImplementedError("write your pallas kernel here")


# speedup vs baseline: 4.4763x; 4.4763x over previous
import functools

import jax
import jax.numpy as jnp
from jax import lax
from jax.experimental import pallas as pl
from jax.experimental.pallas import tpu as pltpu
from jax.experimental.pallas import tpu_sc as plsc

N = 10000
E = 320000
C = 128
NL = 48

NC = 2
NS = 16
NW = NC * NS
K = 128
CHUNKS = E // K
NP = 10240
ROWS_PER_SUB = NP // NS
ZCOPY = 128

_mesh = plsc.VectorSubcoreMesh(core_axis_name="c", subcore_axis_name="s")


@functools.partial(
    pl.kernel,
    mesh=_mesh,
    out_type=jax.ShapeDtypeStruct((NC * NP, C), jnp.float32),
    scratch_types=[
        pltpu.VMEM((K,), jnp.int32),
        pltpu.VMEM((K,), jnp.int32),
        pltpu.VMEM((K,), jnp.float32),
        pltpu.VMEM((K, C), jnp.float32),
        pltpu.VMEM_SHARED((NP, C), jnp.float32),
        pltpu.SemaphoreType.DMA,
    ],
)
def _sc_edge_pass(h_hbm, src_hbm, dst_hbm, w_hbm, out_hbm,
                  src_v, dst_v, w_v, rows_v, agg_sh, sem):
    c = lax.axis_index("c")
    s = lax.axis_index("s")
    wid = s * NC + c

    def _zero_row(i, carry):
        for j in range(C // 16):
            rows_v[i, pl.ds(j * 16, 16)] = jnp.zeros((16,), jnp.float32)
        return carry
    lax.fori_loop(0, ZCOPY, _zero_row, 0)
    for k in range(ROWS_PER_SUB // ZCOPY):
        pltpu.sync_copy(rows_v.at[pl.ds(0, ZCOPY)],
                        agg_sh.at[pl.ds(s * ROWS_PER_SUB + k * ZCOPY, ZCOPY)])
    plsc.subcore_barrier()

    ntrips = (CHUNKS - wid + NW - 1) // NW

    def _chunk(t, carry):
        base = (wid + t * NW) * K
        pltpu.sync_copy(src_hbm.at[pl.ds(base, K)], src_v)
        pltpu.sync_copy(dst_hbm.at[pl.ds(base, K)], dst_v)
        pltpu.sync_copy(w_hbm.at[pl.ds(base, K)], w_v)
        pltpu.async_copy(h_hbm.at[src_v], rows_v, sem).wait()

        def _scale_group(g, cc):
            w16 = w_v[pl.ds(g * 16, 16)]
            for l in range(16):
                wvec = lax.broadcast_in_dim(w16[l], (16,), ())
                i = g * 16 + l
                for j in range(C // 16):
                    sl = pl.ds(j * 16, 16)
                    rows_v[i, sl] = rows_v[i, sl] * wvec
            return cc
        lax.fori_loop(0, K // 16, _scale_group, 0)

        pltpu.sync_copy(rows_v, agg_sh.at[dst_v], add=True)
        return carry

    lax.fori_loop(0, ntrips, _chunk, 0)
    plsc.subcore_barrier()

    for k in range(ROWS_PER_SUB // ZCOPY):
        r0 = s * ROWS_PER_SUB + k * ZCOPY
        pltpu.sync_copy(agg_sh.at[pl.ds(r0, ZCOPY)],
                        out_hbm.at[pl.ds(c * NP + r0, ZCOPY)])


_BR = 1000


def _tc_pre(x, W1, W2):
    def body(x_ref, w1_ref, w2_ref, h_ref, skip_ref):
        xb = x_ref[...]
        h_ref[...] = jnp.dot(xb, w1_ref[...], preferred_element_type=jnp.float32)
        skip_ref[...] = jnp.dot(xb, w2_ref[...], preferred_element_type=jnp.float32)

    return pl.pallas_call(
        body,
        grid=(N // _BR,),
        in_specs=[
            pl.BlockSpec((_BR, C), lambda i: (i, 0)),
            pl.BlockSpec((C, C), lambda i: (0, 0)),
            pl.BlockSpec((C, C), lambda i: (0, 0)),
        ],
        out_specs=[
            pl.BlockSpec((_BR, C), lambda i: (i, 0)),
            pl.BlockSpec((_BR, C), lambda i: (i, 0)),
        ],
        out_shape=[
            jax.ShapeDtypeStruct((N, C), jnp.float32),
            jax.ShapeDtypeStruct((N, C), jnp.float32),
        ],
    )(x, W1, W2)


def _tc_mid(p0, p1, skip, b, W1, W2):
    def body(p0_ref, p1_ref, skip_ref, b_ref, w1_ref, w2_ref, h_ref, skip2_ref):
        t = jax.nn.relu(p0_ref[...] + p1_ref[...] + skip_ref[...] + b_ref[...])
        h_ref[...] = jnp.dot(t, w1_ref[...], preferred_element_type=jnp.float32)
        skip2_ref[...] = jnp.dot(t, w2_ref[...], preferred_element_type=jnp.float32)

    return pl.pallas_call(
        body,
        grid=(N // _BR,),
        in_specs=[
            pl.BlockSpec((_BR, C), lambda i: (i, 0)),
            pl.BlockSpec((_BR, C), lambda i: (i, 0)),
            pl.BlockSpec((_BR, C), lambda i: (i, 0)),
            pl.BlockSpec((1, C), lambda i: (0, 0)),
            pl.BlockSpec((C, C), lambda i: (0, 0)),
            pl.BlockSpec((C, C), lambda i: (0, 0)),
        ],
        out_specs=[
            pl.BlockSpec((_BR, C), lambda i: (i, 0)),
            pl.BlockSpec((_BR, C), lambda i: (i, 0)),
        ],
        out_shape=[
            jax.ShapeDtypeStruct((N, C), jnp.float32),
            jax.ShapeDtypeStruct((N, C), jnp.float32),
        ],
    )(p0, p1, skip, b, W1, W2)


def _tc_final(p0, p1, skip, b, Wd, bd):
    def body(p0_ref, p1_ref, skip_ref, b_ref, wd_ref, bd_ref, o_ref):
        t = jax.nn.relu(p0_ref[...] + p1_ref[...] + skip_ref[...] + b_ref[...])
        o_ref[...] = jnp.dot(t, wd_ref[...], preferred_element_type=jnp.float32) + bd_ref[...]

    return pl.pallas_call(
        body,
        grid=(N // _BR,),
        in_specs=[
            pl.BlockSpec((_BR, C), lambda i: (i, 0)),
            pl.BlockSpec((_BR, C), lambda i: (i, 0)),
            pl.BlockSpec((_BR, C), lambda i: (i, 0)),
            pl.BlockSpec((1, C), lambda i: (0, 0)),
            pl.BlockSpec((C, NL), lambda i: (0, 0)),
            pl.BlockSpec((1, NL), lambda i: (0, 0)),
        ],
        out_specs=pl.BlockSpec((_BR, NL), lambda i: (i, 0)),
        out_shape=jax.ShapeDtypeStruct((N, NL), jnp.float32),
    )(p0, p1, skip, b, Wd, bd)


def kernel(x, edge_index, edge_weight, W1_1, W2_1, b1, W1_2, W2_2, b2, Wd, bd):
    src = edge_index[0].astype(jnp.int32)
    dst = edge_index[1].astype(jnp.int32)
    w = edge_weight.astype(jnp.float32)
    b1r = b1.reshape(1, C)
    b2r = b2.reshape(1, C)
    bdr = bd.reshape(1, NL)

    h, skip = _tc_pre(x, W1_1, W2_1)
    p = _sc_edge_pass(h, src, dst, w)
    h, skip = _tc_mid(p[:N], p[NP:NP + N], skip, b1r, W1_2, W2_2)
    p = _sc_edge_pass(h, src, dst, w)
    h, skip = _tc_mid(p[:N], p[NP:NP + N], skip, b2r, W1_2, W2_2)
    p = _sc_edge_pass(h, src, dst, w)
    return _tc_final(p[:N], p[NP:NP + N], skip, b2r, Wd, bdr)


# speedup vs baseline: 9.4093x; 2.1020x over previous
import functools

import jax
import jax.numpy as jnp
from jax import lax
from jax.experimental import pallas as pl
from jax.experimental.pallas import tpu as pltpu
from jax.experimental.pallas import tpu_sc as plsc

N = 10000
E = 320000
C = 128
NL = 48

NC = 2
NS = 16
NW = NC * NS
K = 128
WPT = 80
BLK = 16
NBLK = WPT // BLK
NCHUNK = NW * WPT
E_PAD = NCHUNK * K
NP = 10240
ROWS_PER_SUB = NP // NS

_mesh = plsc.VectorSubcoreMesh(core_axis_name="c", subcore_axis_name="s")


@functools.partial(
    pl.kernel,
    mesh=_mesh,
    out_type=jax.ShapeDtypeStruct((NC * NP, C), jnp.float32),
    scratch_types=[
        pltpu.VMEM((2, BLK, K), jnp.int32),
        pltpu.VMEM((2, BLK, K), jnp.int32),
        pltpu.VMEM((2, BLK, K), jnp.float32),
        pltpu.VMEM((K, C), jnp.float32),
        pltpu.VMEM((K, C), jnp.float32),
        pltpu.VMEM_SHARED((NP, C), jnp.float32),
        pltpu.SemaphoreType.DMA,
        pltpu.SemaphoreType.DMA,
        pltpu.SemaphoreType.DMA,
        pltpu.SemaphoreType.DMA,
    ],
)
def _sc_edge_pass(h_hbm, src_hbm, dst_hbm, w_hbm, out_hbm,
                  srcb, dstb, wb, rows_a, rows_b, agg_sh,
                  semi0, semi1, sema, semb):
    c = lax.axis_index("c")
    s = lax.axis_index("s")
    wid = s * NC + c
    semi = (semi0, semi1)

    def _idx_block_start(b, st):
        pltpu.async_copy(src_hbm.at[wid, b], srcb.at[st], semi[st])
        pltpu.async_copy(dst_hbm.at[wid, b], dstb.at[st], semi[st])
        pltpu.async_copy(w_hbm.at[wid, b], wb.at[st], semi[st])

    def _idx_block_wait(b, st):
        pltpu.make_async_copy(src_hbm.at[wid, b], srcb.at[st], semi[st]).wait()
        pltpu.make_async_copy(dst_hbm.at[wid, b], dstb.at[st], semi[st]).wait()
        pltpu.make_async_copy(w_hbm.at[wid, b], wb.at[st], semi[st]).wait()

    _idx_block_start(0, 0)

    def _zero_row(i, carry):
        for j in range(C // 16):
            rows_a[i, pl.ds(j * 16, 16)] = jnp.zeros((16,), jnp.float32)
        return carry
    lax.fori_loop(0, K, _zero_row, 0)
    for k in range(ROWS_PER_SUB // K):
        pltpu.sync_copy(rows_a,
                        agg_sh.at[pl.ds(s * ROWS_PER_SUB + k * K, K)])
    plsc.subcore_barrier()

    def _scale(rows, wset, t):
        def _group(g, cc):
            w16 = wset[t, pl.ds(g * 16, 16)]
            for l in range(16):
                wvec = lax.broadcast_in_dim(w16[l], (16,), ())
                i = g * 16 + l
                for j in range(C // 16):
                    sl = pl.ds(j * 16, 16)
                    rows[i, sl] = rows[i, sl] * wvec
            return cc
        lax.fori_loop(0, K // 16, _group, 0)

    _idx_block_wait(0, 0)
    pltpu.async_copy(h_hbm.at[srcb.at[0, 0]], rows_a, sema)

    for b in range(NBLK):
        st = b % 2
        if b + 1 < NBLK:
            _idx_block_start(b + 1, 1 - st)
        src_s, dst_s, w_s = srcb.at[st], dstb.at[st], wb.at[st]

        def _pair(hh, carry):
            t0 = hh * 2
            t1 = t0 + 1
            pltpu.make_async_copy(h_hbm.at[src_s.at[t0]], rows_a, sema).wait()
            pltpu.async_copy(h_hbm.at[src_s.at[t1]], rows_b, semb)
            _scale(rows_a, w_s, t0)
            pltpu.sync_copy(rows_a, agg_sh.at[dst_s.at[t0]], add=True)
            pltpu.make_async_copy(h_hbm.at[src_s.at[t1]], rows_b, semb).wait()

            @pl.when(hh < BLK // 2 - 1)
            def _():
                pltpu.async_copy(h_hbm.at[src_s.at[t0 + 2]], rows_a, sema)
            _scale(rows_b, w_s, t1)
            pltpu.sync_copy(rows_b, agg_sh.at[dst_s.at[t1]], add=True)
            return carry

        lax.fori_loop(0, BLK // 2, _pair, 0)
        if b + 1 < NBLK:
            _idx_block_wait(b + 1, 1 - st)
            pltpu.async_copy(h_hbm.at[srcb.at[1 - st, 0]], rows_a, sema)

    plsc.subcore_barrier()

    for k in range(ROWS_PER_SUB // K):
        r0 = s * ROWS_PER_SUB + k * K
        pltpu.sync_copy(agg_sh.at[pl.ds(r0, K)],
                        out_hbm.at[pl.ds(c * NP + r0, K)])


_BR = 1000


def _tc_pre(x, W1, W2):
    def body(x_ref, w1_ref, w2_ref, h_ref, skip_ref):
        xb = x_ref[...]
        h_ref[...] = jnp.dot(xb, w1_ref[...], preferred_element_type=jnp.float32)
        skip_ref[...] = jnp.dot(xb, w2_ref[...], preferred_element_type=jnp.float32)

    return pl.pallas_call(
        body,
        grid=(N // _BR,),
        in_specs=[
            pl.BlockSpec((_BR, C), lambda i: (i, 0)),
            pl.BlockSpec((C, C), lambda i: (0, 0)),
            pl.BlockSpec((C, C), lambda i: (0, 0)),
        ],
        out_specs=[
            pl.BlockSpec((_BR, C), lambda i: (i, 0)),
            pl.BlockSpec((_BR, C), lambda i: (i, 0)),
        ],
        out_shape=[
            jax.ShapeDtypeStruct((N, C), jnp.float32),
            jax.ShapeDtypeStruct((N, C), jnp.float32),
        ],
    )(x, W1, W2)


def _tc_mid(p0, p1, skip, b, W1, W2):
    def body(p0_ref, p1_ref, skip_ref, b_ref, w1_ref, w2_ref, h_ref, skip2_ref):
        t = jax.nn.relu(p0_ref[...] + p1_ref[...] + skip_ref[...] + b_ref[...])
        h_ref[...] = jnp.dot(t, w1_ref[...], preferred_element_type=jnp.float32)
        skip2_ref[...] = jnp.dot(t, w2_ref[...], preferred_element_type=jnp.float32)

    return pl.pallas_call(
        body,
        grid=(N // _BR,),
        in_specs=[
            pl.BlockSpec((_BR, C), lambda i: (i, 0)),
            pl.BlockSpec((_BR, C), lambda i: (i, 0)),
            pl.BlockSpec((_BR, C), lambda i: (i, 0)),
            pl.BlockSpec((1, C), lambda i: (0, 0)),
            pl.BlockSpec((C, C), lambda i: (0, 0)),
            pl.BlockSpec((C, C), lambda i: (0, 0)),
        ],
        out_specs=[
            pl.BlockSpec((_BR, C), lambda i: (i, 0)),
            pl.BlockSpec((_BR, C), lambda i: (i, 0)),
        ],
        out_shape=[
            jax.ShapeDtypeStruct((N, C), jnp.float32),
            jax.ShapeDtypeStruct((N, C), jnp.float32),
        ],
    )(p0, p1, skip, b, W1, W2)


def _tc_final(p0, p1, skip, b, Wd, bd):
    def body(p0_ref, p1_ref, skip_ref, b_ref, wd_ref, bd_ref, o_ref):
        t = jax.nn.relu(p0_ref[...] + p1_ref[...] + skip_ref[...] + b_ref[...])
        o_ref[...] = jnp.dot(t, wd_ref[...], preferred_element_type=jnp.float32) + bd_ref[...]

    return pl.pallas_call(
        body,
        grid=(N // _BR,),
        in_specs=[
            pl.BlockSpec((_BR, C), lambda i: (i, 0)),
            pl.BlockSpec((_BR, C), lambda i: (i, 0)),
            pl.BlockSpec((_BR, C), lambda i: (i, 0)),
            pl.BlockSpec((1, C), lambda i: (0, 0)),
            pl.BlockSpec((C, NL), lambda i: (0, 0)),
            pl.BlockSpec((1, NL), lambda i: (0, 0)),
        ],
        out_specs=pl.BlockSpec((_BR, NL), lambda i: (i, 0)),
        out_shape=jax.ShapeDtypeStruct((N, NL), jnp.float32),
    )(p0, p1, skip, b, Wd, bd)


def kernel(x, edge_index, edge_weight, W1_1, W2_1, b1, W1_2, W2_2, b2, Wd, bd):
    npad = E_PAD - E
    pad_idx = (jnp.arange(npad, dtype=jnp.int32) * 13) % N
    src = jnp.concatenate([edge_index[0].astype(jnp.int32), pad_idx]).reshape(NW, NBLK, BLK, K)
    dst = jnp.concatenate([edge_index[1].astype(jnp.int32), pad_idx]).reshape(NW, NBLK, BLK, K)
    w = jnp.concatenate([edge_weight.astype(jnp.float32),
                         jnp.zeros((npad,), jnp.float32)]).reshape(NW, NBLK, BLK, K)
    b1r = b1.reshape(1, C)
    b2r = b2.reshape(1, C)
    bdr = bd.reshape(1, NL)

    h, skip = _tc_pre(x, W1_1, W2_1)
    p = _sc_edge_pass(h, src, dst, w)
    h, skip = _tc_mid(p[:N], p[NP:NP + N], skip, b1r, W1_2, W2_2)
    p = _sc_edge_pass(h, src, dst, w)
    h, skip = _tc_mid(p[:N], p[NP:NP + N], skip, b2r, W1_2, W2_2)
    p = _sc_edge_pass(h, src, dst, w)
    return _tc_final(p[:N], p[NP:NP + N], skip, b2r, Wd, bdr)
